# baseline (speedup 1.0000x reference)
import functools

import jax
import jax.numpy as jnp
from jax import lax
from jax.experimental import pallas as pl
from jax.experimental.pallas import tpu as pltpu
from jax.experimental.pallas import tpu_sc as plsc

F32 = jnp.float32
BF16 = jnp.bfloat16


def _elu(v):
    return jnp.where(v > 0, v, jnp.exp(v) - 1.0)


def _pick_tile(n, candidates):
    for t in candidates:
        if n % t == 0:
            return t
    return n


def _edge_body(pre_ref, e_ref, ew_ref, emb_ref, embW_ref, embb_ref,
               wa_ref, wb_ref, wc_ref, b0_ref, w1_ref, b1_ref,
               w2_ref, b2_ref, w3_ref, b3_ref, out_ref):
    c = (jnp.dot(emb_ref[...], embW_ref[...],
                 preferred_element_type=F32) + embb_ref[...]).astype(BF16)
    b0eff = (b0_ref[...]
             + jnp.dot(c, wa_ref[...], preferred_element_type=F32)
             + jnp.dot(c, wb_ref[...], preferred_element_type=F32))
    e = e_ref[...]
    h = (pre_ref[...]
         + jnp.dot(e.astype(BF16), wc_ref[...], preferred_element_type=F32)
         + b0eff)
    h = _elu(h)
    h = _elu(jnp.dot(h.astype(BF16), w1_ref[...],
                     preferred_element_type=F32) + b1_ref[...])
    h = _elu(jnp.dot(h.astype(BF16), w2_ref[...],
                     preferred_element_type=F32) + b2_ref[...])
    h = jnp.dot(h.astype(BF16), w3_ref[...],
                preferred_element_type=F32) + b3_ref[...]
    out_ref[...] = (e + h) * ew_ref[...]


_EDGE_TILES = (2560, 2000, 1280, 1000, 640, 512, 256, 128, 64, 32, 16, 8)


def _edge_body_chained(pre_ref, e_ref, ew_ref, emb_ref, embW_ref, embb_ref,
                       wa_ref, wb_ref, wc_ref, b0_ref, w1_ref, b1_ref,
                       w2_ref, b2_ref, w3_ref, b3_ref, prev_ref, out_ref):
    del prev_ref
    _edge_body(pre_ref, e_ref, ew_ref, emb_ref, embW_ref, embb_ref,
               wa_ref, wb_ref, wc_ref, b0_ref, w1_ref, b1_ref,
               w2_ref, b2_ref, w3_ref, b3_ref, out_ref)


def _edge_mlp_slice(pre, e, ew2, emb, emb_W, emb_b, eW0, eb0, eWh, ebh,
                    off, prev):
    E, C = e.shape
    n = pre.shape[0]
    T = _pick_tile(n, _EDGE_TILES)
    ob = off // T
    wa = eW0[0 * C:1 * C].astype(BF16)
    wb = eW0[1 * C:2 * C].astype(BF16)
    wc = eW0[2 * C:3 * C].astype(BF16)
    w1, w2, w3 = (eWh[i].astype(BF16) for i in range(3))
    b0 = eb0.reshape(1, C)
    b1, b2, b3 = (ebh[i].reshape(1, C) for i in range(3))

    off_spec = pl.BlockSpec((T, C), lambda i: (i + ob, 0))
    full = lambda a: pl.BlockSpec(a.shape, lambda i: (0,) * a.ndim)
    body = _edge_body if prev is None else _edge_body_chained
    in_specs = [pl.BlockSpec((T, C), lambda i: (i, 0)), off_spec,
                pl.BlockSpec((T, 1), lambda i: (i + ob, 0)),
                full(emb), full(emb_W), full(b0),
                full(wa), full(wb), full(wc), full(b0),
                full(w1), full(b1), full(w2), full(b2), full(w3), full(b3)]
    args = [pre, e, ew2, emb, emb_W, emb_b.reshape(1, C),
            wa, wb, wc, b0, w1, b1, w2, b2, w3, b3]
    kwargs = {}
    if prev is not None:
        in_specs.append(pl.BlockSpec((8, C), lambda i: (0, 0)))
        args.append(prev)
        kwargs["input_output_aliases"] = {len(args) - 1: 0}
    return pl.pallas_call(
        body,
        grid=(n // T,),
        in_specs=in_specs,
        out_specs=off_spec,
        out_shape=jax.ShapeDtypeStruct((E, C), F32),
        compiler_params=pltpu.CompilerParams(
            dimension_semantics=("arbitrary",)),
        **kwargs,
    )(*args)


def _uv_body(x_ref, wa_ref, wb_ref, u_ref, v_ref):
    xb = x_ref[...].astype(BF16)
    u_ref[...] = jnp.dot(xb, wa_ref[...], preferred_element_type=F32)
    v_ref[...] = jnp.dot(xb, wb_ref[...], preferred_element_type=F32)


def _uv_project(x, eW0):
    N, C = x.shape
    T = _pick_tile(N, (2000, 1000, 500, 200, 100, 50, 40, 25, 16, 8))
    wa = eW0[0 * C:1 * C].astype(BF16)
    wb = eW0[1 * C:2 * C].astype(BF16)
    row_spec = pl.BlockSpec((T, C), lambda i: (i, 0))
    full = lambda a: pl.BlockSpec(a.shape, lambda i: (0,) * a.ndim)
    return pl.pallas_call(
        _uv_body,
        grid=(N // T,),
        in_specs=[row_spec, full(wa), full(wb)],
        out_specs=[row_spec, row_spec],
        out_shape=[jax.ShapeDtypeStruct((N, C), F32),
                   jax.ShapeDtypeStruct((N, C), F32)],
        compiler_params=pltpu.CompilerParams(
            dimension_semantics=("arbitrary",)),
    )(x, wa, wb)


_KG = 2


def _sc_gather_sum(u, v, src1d, dst1d):
    N, CW = u.shape
    E = src1d.shape[0]
    rows = _KG * _CHUNK
    tot_groups = E // rows
    smax = (tot_groups + _NW - 1) // _NW
    t3 = (smax + 3) // 3
    mesh = plsc.VectorSubcoreMesh(core_axis_name="c", subcore_axis_name="s")
    nbuf = 3

    @functools.partial(
        pl.kernel,
        out_type=jax.ShapeDtypeStruct((E, CW), F32),
        mesh=mesh,
        scratch_types=[
            [pltpu.VMEM((rows,), jnp.int32) for _ in range(nbuf)],
            [pltpu.VMEM((rows,), jnp.int32) for _ in range(nbuf)],
            [pltpu.VMEM((rows, CW), F32) for _ in range(nbuf)],
            [pltpu.SemaphoreType.DMA for _ in range(nbuf)],
            [pltpu.SemaphoreType.DMA for _ in range(nbuf)],
            [pltpu.SemaphoreType.DMA for _ in range(nbuf)],
        ],
    )
    def gat(u_hbm, v_hbm, s_hbm, d_hbm, pre_hbm, si_v, di_v, r_v,
            semu, semv, semw):
        cid = lax.axis_index("c")
        sid = lax.axis_index("s")
        wid = sid * _NC + cid

        def drain(sem, dst_ref):
            pltpu.make_async_copy(u_hbm.at[pl.ds(0, rows)], dst_ref,
                                  sem).wait()

        def w_drain(sem):
            pltpu.make_async_copy(r_v[0], pre_hbm.at[pl.ds(0, rows)],
                                  sem).wait()

        def body(t, carry):
            for b in range(nbuf):
                s = t * nbuf + b
                g = wid + s * _NW
                bp = (b + nbuf - 1) % nbuf

                @pl.when((g < tot_groups) & (t >= 1))
                def _():
                    w_drain(semw[b])

                @pl.when(g < tot_groups)
                def _():
                    base = g * rows
                    pltpu.sync_copy(s_hbm.at[pl.ds(base, rows)], si_v[b])
                    pltpu.sync_copy(d_hbm.at[pl.ds(base, rows)], di_v[b])
                    for j in range(_KG):
                        sl = pl.ds(j * _CHUNK, _CHUNK)
                        pltpu.async_copy(u_hbm.at[si_v[b].at[sl]],
                                         r_v[b].at[sl], semu[b])

                gp = wid + (s - 1) * _NW
                cond = (gp < tot_groups) if b != 0 else (
                    (gp < tot_groups) & (t >= 1))

                @pl.when(cond)
                def _():
                    basep = gp * rows
                    drain(semu[bp], r_v[bp])
                    descs = []
                    for j in range(_KG):
                        sl = pl.ds(j * _CHUNK, _CHUNK)
                        descs.append(pltpu.async_copy(
                            v_hbm.at[di_v[bp].at[sl]], r_v[bp].at[sl],
                            semv[bp], add=True))
                    for dsc in descs:
                        dsc.wait()
                    pltpu.async_copy(r_v[bp], pre_hbm.at[pl.ds(basep, rows)],
                                     semw[bp])
            return carry

        lax.fori_loop(0, t3, body, 0)
        for b in range(nbuf):
            w_drain(semw[b])

    return gat(u, v, src1d, dst1d)


_NC, _NS = 2, 16
_NW = _NC * _NS
_CHUNK = 128
_K = 2


def _sc_scatter_add(e2d, dst3d, zeros_nc):
    E, C = e2d.shape
    N = zeros_nc.shape[0]
    tot_groups = E // (_CHUNK * _K)
    outer = (tot_groups + _NW - 1) // _NW
    rpt = (N // _NS) & ~7
    tail = N - _NS * rpt
    mesh = plsc.VectorSubcoreMesh(core_axis_name="c", subcore_axis_name="s")

    @functools.partial(
        pl.kernel,
        out_type=jax.ShapeDtypeStruct((_NC, N, C), F32),
        mesh=mesh,
        scratch_types=[
            pltpu.VMEM((_K, _CHUNK), jnp.int32),
            pltpu.VMEM((_K * _CHUNK, C), F32),
            pltpu.VMEM_SHARED((N, C), F32),
            pltpu.SemaphoreType.DMA,
        ],
    )
    def scat(e_hbm, dst_hbm, zero_hbm, out_hbm, idx_v, rows_v, acc_sh, sem):
        cid = lax.axis_index("c")
        sid = lax.axis_index("s")
        wid = sid * _NC + cid
        pltpu.sync_copy(zero_hbm.at[pl.ds(sid * rpt, rpt)],
                        acc_sh.at[pl.ds(sid * rpt, rpt)])
        if tail:
            @pl.when(sid == _NS - 1)
            def _():
                pltpu.sync_copy(zero_hbm.at[pl.ds(_NS * rpt, tail)],
                                acc_sh.at[pl.ds(_NS * rpt, tail)])
        plsc.subcore_barrier()

        def body(t, carry):
            g = wid + t * _NW

            @pl.when(g < tot_groups)
            def _():
                pltpu.sync_copy(dst_hbm.at[g], idx_v)
                pltpu.sync_copy(e_hbm.at[pl.ds(g * _K * _CHUNK, _K * _CHUNK)],
                                rows_v)
                descs = []
                for j in range(_K):
                    descs.append(pltpu.async_copy(
                        rows_v.at[pl.ds(j * _CHUNK, _CHUNK)],
                        acc_sh.at[idx_v.at[j]], sem, add=True))
                for d in descs:
                    d.wait()
            return carry

        lax.fori_loop(0, outer, body, 0)
        plsc.subcore_barrier()
        pltpu.sync_copy(
            acc_sh.at[pl.ds(sid * rpt, rpt)],
            out_hbm.at[cid, pl.ds(sid * rpt, rpt)])
        if tail:
            @pl.when(sid == _NS - 1)
            def _():
                pltpu.sync_copy(
                    acc_sh.at[pl.ds(_NS * rpt, tail)],
                    out_hbm.at[cid, pl.ds(_NS * rpt, tail)])

    return scat(e2d, dst3d, zeros_nc)


def _node_body(x_ref, agg_ref, agg2_ref, emb_ref, embW_ref, embb_ref,
               wa_ref, wb_ref, b0_ref, w1_ref, b1_ref,
               w2_ref, b2_ref, w3_ref, b3_ref, out_ref):
    c = jnp.dot(emb_ref[...], embW_ref[...],
                preferred_element_type=F32) + embb_ref[...]
    xc = x_ref[...] + c
    agg = agg_ref[...] + agg2_ref[...]
    h = (jnp.dot(xc.astype(BF16), wa_ref[...], preferred_element_type=F32)
         + jnp.dot(agg.astype(BF16), wb_ref[...], preferred_element_type=F32)
         + b0_ref[...])
    h = _elu(h)
    h = _elu(jnp.dot(h.astype(BF16), w1_ref[...],
                     preferred_element_type=F32) + b1_ref[...])
    h = _elu(jnp.dot(h.astype(BF16), w2_ref[...],
                     preferred_element_type=F32) + b2_ref[...])
    h = jnp.dot(h.astype(BF16), w3_ref[...],
                preferred_element_type=F32) + b3_ref[...]
    out_ref[...] = xc + h


def _node_mlp(x, agg, agg2, emb, emb_W, emb_b, nW0, nb0, nWh, nbh):
    N, C = x.shape
    T = _pick_tile(N, (2000, 1000, 500, 200, 100, 50, 40, 25, 16, 8))
    wa = nW0[0 * C:1 * C].astype(BF16)
    wb = nW0[1 * C:2 * C].astype(BF16)
    w1, w2, w3 = (nWh[i].astype(BF16) for i in range(3))
    b0 = nb0.reshape(1, C)
    b1, b2, b3 = (nbh[i].reshape(1, C) for i in range(3))

    row_spec = pl.BlockSpec((T, C), lambda i: (i, 0))
    full = lambda a: pl.BlockSpec(a.shape, lambda i: (0,) * a.ndim)
    return pl.pallas_call(
        _node_body,
        grid=(N // T,),
        in_specs=[row_spec, row_spec, row_spec,
                  full(emb), full(emb_W), full(b0),
                  full(wa), full(wb), full(b0),
                  full(w1), full(b1), full(w2), full(b2), full(w3), full(b3)],
        out_specs=row_spec,
        out_shape=jax.ShapeDtypeStruct((N, C), F32),
        compiler_params=pltpu.CompilerParams(
            dimension_semantics=("arbitrary",)),
    )(x, agg, agg2, emb, emb_W, emb_b.reshape(1, C),
      wa, wb, b0, w1, b1, w2, b2, w3, b3)


def kernel(x, e, emb, edge_index, edge_weight, halo_info, mask_send,
           mask_recv, buffer_send, buffer_recv, neighboring_procs, SIZE,
           emb_W, emb_b, eW0, eb0, eWh, ebh, nW0, nb0, nWh, nbh):
    src = edge_index[0]
    dst = edge_index[1]
    E = e.shape[0]
    N, C = x.shape
    u, v = _uv_project(x, eW0)
    ew2 = edge_weight.reshape(E, 1)
    E2 = E // 2
    if E2 % (_KG * _CHUNK) == 0 and E2 % _pick_tile(E2, _EDGE_TILES) == 0:
        pre1 = _sc_gather_sum(u, v, src[:E2], dst[:E2])
        pre2 = _sc_gather_sum(u, v, src[E2:], dst[E2:])
        eo1 = _edge_mlp_slice(pre1, e, ew2, emb, emb_W, emb_b,
                              eW0, eb0, eWh, ebh, 0, None)
        e_out = _edge_mlp_slice(pre2, e, ew2, emb, emb_W, emb_b,
                                eW0, eb0, eWh, ebh, E2, eo1)
    else:
        pre = _sc_gather_sum(u, v, src, dst)
        e_out = _edge_mlp_slice(pre, e, ew2, emb, emb_W, emb_b,
                                eW0, eb0, eWh, ebh, 0, None)
    if E % (_CHUNK * _K) == 0:
        dst3d = dst.reshape(E // (_CHUNK * _K), _K, _CHUNK)
        zeros_nc = jnp.zeros((N, C), F32)
        partials = _sc_scatter_add(e_out, dst3d, zeros_nc)
        agg, agg2 = partials[0], partials[1]
    else:
        agg = jax.ops.segment_sum(e_out, dst, num_segments=N)
        agg2 = jnp.zeros((N, C), F32)
    x_out = _node_mlp(x, agg, agg2, emb, emb_W, emb_b, nW0, nb0, nWh, nbh)
    return (x_out, e_out)


# speedup vs baseline: 1.0567x; 1.0567x over previous
import functools

import jax
import jax.numpy as jnp
from jax import lax
from jax.experimental import pallas as pl
from jax.experimental.pallas import tpu as pltpu
from jax.experimental.pallas import tpu_sc as plsc

F32 = jnp.float32
BF16 = jnp.bfloat16


def _elu(v):
    return jnp.where(v > 0, v, jnp.exp(v) - 1.0)


def _pick_tile(n, candidates):
    for t in candidates:
        if n % t == 0:
            return t
    return n


def _edge_body(pre_ref, e_ref, ew_ref, emb_ref, embW_ref, embb_ref,
               wa_ref, wb_ref, wc_ref, b0_ref, w1_ref, b1_ref,
               w2_ref, b2_ref, w3_ref, b3_ref, out_ref):
    c = (jnp.dot(emb_ref[...], embW_ref[...],
                 preferred_element_type=F32) + embb_ref[...]).astype(BF16)
    b0eff = (b0_ref[...]
             + jnp.dot(c, wa_ref[...], preferred_element_type=F32)
             + jnp.dot(c, wb_ref[...], preferred_element_type=F32))
    e = e_ref[...]
    h = (pre_ref[...]
         + jnp.dot(e.astype(BF16), wc_ref[...], preferred_element_type=F32)
         + b0eff)
    h = _elu(h)
    h = _elu(jnp.dot(h.astype(BF16), w1_ref[...],
                     preferred_element_type=F32) + b1_ref[...])
    h = _elu(jnp.dot(h.astype(BF16), w2_ref[...],
                     preferred_element_type=F32) + b2_ref[...])
    h = jnp.dot(h.astype(BF16), w3_ref[...],
                preferred_element_type=F32) + b3_ref[...]
    out_ref[...] = (e + h) * jnp.transpose(ew_ref[...])


_EDGE_TILES = (2560, 1280, 640, 512, 256, 128)


def _edge_body_chained(pre_ref, e_ref, ew_ref, emb_ref, embW_ref, embb_ref,
                       wa_ref, wb_ref, wc_ref, b0_ref, w1_ref, b1_ref,
                       w2_ref, b2_ref, w3_ref, b3_ref, prev_ref, out_ref):
    del prev_ref
    _edge_body(pre_ref, e_ref, ew_ref, emb_ref, embW_ref, embb_ref,
               wa_ref, wb_ref, wc_ref, b0_ref, w1_ref, b1_ref,
               w2_ref, b2_ref, w3_ref, b3_ref, out_ref)


def _edge_mlp_slice(pre, e, ew2, emb, emb_W, emb_b, eW0, eb0, eWh, ebh,
                    off, prev):
    E, C = e.shape
    n = pre.shape[0]
    T = _pick_tile(n, _EDGE_TILES)
    ob = off // T
    wa = eW0[0 * C:1 * C].astype(BF16)
    wb = eW0[1 * C:2 * C].astype(BF16)
    wc = eW0[2 * C:3 * C].astype(BF16)
    w1, w2, w3 = (eWh[i].astype(BF16) for i in range(3))
    b0 = eb0.reshape(1, C)
    b1, b2, b3 = (ebh[i].reshape(1, C) for i in range(3))

    off_spec = pl.BlockSpec((T, C), lambda i: (i + ob, 0))
    full = lambda a: pl.BlockSpec(a.shape, lambda i: (0,) * a.ndim)
    body = _edge_body if prev is None else _edge_body_chained
    in_specs = [pl.BlockSpec((T, C), lambda i: (i, 0)), off_spec,
                pl.BlockSpec((1, T), lambda i: (0, i + ob)),
                full(emb), full(emb_W), full(b0),
                full(wa), full(wb), full(wc), full(b0),
                full(w1), full(b1), full(w2), full(b2), full(w3), full(b3)]
    args = [pre, e, ew2, emb, emb_W, emb_b.reshape(1, C),
            wa, wb, wc, b0, w1, b1, w2, b2, w3, b3]
    kwargs = {}
    if prev is not None:
        in_specs.append(pl.BlockSpec((8, C), lambda i: (0, 0)))
        args.append(prev)
        kwargs["input_output_aliases"] = {len(args) - 1: 0}
    return pl.pallas_call(
        body,
        grid=(n // T,),
        in_specs=in_specs,
        out_specs=off_spec,
        out_shape=jax.ShapeDtypeStruct((E, C), F32),
        compiler_params=pltpu.CompilerParams(
            dimension_semantics=("arbitrary",)),
        **kwargs,
    )(*args)


def _uv_body(x_ref, wa_ref, wb_ref, u_ref, v_ref):
    xb = x_ref[...].astype(BF16)
    u_ref[...] = jnp.dot(xb, wa_ref[...], preferred_element_type=F32)
    v_ref[...] = jnp.dot(xb, wb_ref[...], preferred_element_type=F32)


def _uv_project(x, eW0):
    N, C = x.shape
    T = _pick_tile(N, (2000, 1000, 500, 200, 100, 50, 40, 25, 16, 8))
    wa = eW0[0 * C:1 * C].astype(BF16)
    wb = eW0[1 * C:2 * C].astype(BF16)
    row_spec = pl.BlockSpec((T, C), lambda i: (i, 0))
    full = lambda a: pl.BlockSpec(a.shape, lambda i: (0,) * a.ndim)
    return pl.pallas_call(
        _uv_body,
        grid=(N // T,),
        in_specs=[row_spec, full(wa), full(wb)],
        out_specs=[row_spec, row_spec],
        out_shape=[jax.ShapeDtypeStruct((N, C), F32),
                   jax.ShapeDtypeStruct((N, C), F32)],
        compiler_params=pltpu.CompilerParams(
            dimension_semantics=("arbitrary",)),
    )(x, wa, wb)


_KG = 2


def _sc_gather_sum(u, v, src1d, dst1d):
    N, CW = u.shape
    E = src1d.shape[0]
    rows = _KG * _CHUNK
    tot_groups = E // rows
    smax = (tot_groups + _NW - 1) // _NW
    t3 = (smax + 3) // 3
    mesh = plsc.VectorSubcoreMesh(core_axis_name="c", subcore_axis_name="s")
    nbuf = 3

    @functools.partial(
        pl.kernel,
        out_type=jax.ShapeDtypeStruct((E, CW), F32),
        mesh=mesh,
        scratch_types=[
            [pltpu.VMEM((rows,), jnp.int32) for _ in range(nbuf)],
            [pltpu.VMEM((rows,), jnp.int32) for _ in range(nbuf)],
            [pltpu.VMEM((rows, CW), F32) for _ in range(nbuf)],
            [pltpu.SemaphoreType.DMA for _ in range(nbuf)],
            [pltpu.SemaphoreType.DMA for _ in range(nbuf)],
            [pltpu.SemaphoreType.DMA for _ in range(nbuf)],
        ],
    )
    def gat(u_hbm, v_hbm, s_hbm, d_hbm, pre_hbm, si_v, di_v, r_v,
            semu, semv, semw):
        cid = lax.axis_index("c")
        sid = lax.axis_index("s")
        wid = sid * _NC + cid

        def drain(sem, dst_ref):
            pltpu.make_async_copy(u_hbm.at[pl.ds(0, rows)], dst_ref,
                                  sem).wait()

        def w_drain(sem):
            pltpu.make_async_copy(r_v[0], pre_hbm.at[pl.ds(0, rows)],
                                  sem).wait()

        def body(t, carry):
            for b in range(nbuf):
                s = t * nbuf + b
                g = wid + s * _NW
                bp = (b + nbuf - 1) % nbuf

                @pl.when((g < tot_groups) & (t >= 1))
                def _():
                    w_drain(semw[b])

                @pl.when(g < tot_groups)
                def _():
                    base = g * rows
                    pltpu.sync_copy(s_hbm.at[pl.ds(base, rows)], si_v[b])
                    pltpu.sync_copy(d_hbm.at[pl.ds(base, rows)], di_v[b])
                    for j in range(_KG):
                        sl = pl.ds(j * _CHUNK, _CHUNK)
                        pltpu.async_copy(u_hbm.at[si_v[b].at[sl]],
                                         r_v[b].at[sl], semu[b])

                gp = wid + (s - 1) * _NW
                cond = (gp < tot_groups) if b != 0 else (
                    (gp < tot_groups) & (t >= 1))

                @pl.when(cond)
                def _():
                    basep = gp * rows
                    drain(semu[bp], r_v[bp])
                    descs = []
                    for j in range(_KG):
                        sl = pl.ds(j * _CHUNK, _CHUNK)
                        descs.append(pltpu.async_copy(
                            v_hbm.at[di_v[bp].at[sl]], r_v[bp].at[sl],
                            semv[bp], add=True))
                    for dsc in descs:
                        dsc.wait()
                    pltpu.async_copy(r_v[bp], pre_hbm.at[pl.ds(basep, rows)],
                                     semw[bp])
            return carry

        lax.fori_loop(0, t3, body, 0)
        for b in range(nbuf):
            w_drain(semw[b])

    return gat(u, v, src1d, dst1d)


_NC, _NS = 2, 16
_NW = _NC * _NS
_CHUNK = 128
_K = 2


def _sc_scatter_add(e2d, dst3d, zeros_nc):
    E, C = e2d.shape
    N = zeros_nc.shape[0]
    tot_groups = E // (_CHUNK * _K)
    outer = (tot_groups + _NW - 1) // _NW
    rpt = (N // _NS) & ~7
    tail = N - _NS * rpt
    mesh = plsc.VectorSubcoreMesh(core_axis_name="c", subcore_axis_name="s")

    @functools.partial(
        pl.kernel,
        out_type=jax.ShapeDtypeStruct((_NC, N, C), F32),
        mesh=mesh,
        scratch_types=[
            pltpu.VMEM((_K, _CHUNK), jnp.int32),
            pltpu.VMEM((_K * _CHUNK, C), F32),
            pltpu.VMEM_SHARED((N, C), F32),
            pltpu.SemaphoreType.DMA,
        ],
    )
    def scat(e_hbm, dst_hbm, zero_hbm, out_hbm, idx_v, rows_v, acc_sh, sem):
        cid = lax.axis_index("c")
        sid = lax.axis_index("s")
        wid = sid * _NC + cid
        pltpu.sync_copy(zero_hbm.at[pl.ds(sid * rpt, rpt)],
                        acc_sh.at[pl.ds(sid * rpt, rpt)])
        if tail:
            @pl.when(sid == _NS - 1)
            def _():
                pltpu.sync_copy(zero_hbm.at[pl.ds(_NS * rpt, tail)],
                                acc_sh.at[pl.ds(_NS * rpt, tail)])
        plsc.subcore_barrier()

        def body(t, carry):
            g = wid + t * _NW

            @pl.when(g < tot_groups)
            def _():
                pltpu.sync_copy(dst_hbm.at[g], idx_v)
                pltpu.sync_copy(e_hbm.at[pl.ds(g * _K * _CHUNK, _K * _CHUNK)],
                                rows_v)
                descs = []
                for j in range(_K):
                    descs.append(pltpu.async_copy(
                        rows_v.at[pl.ds(j * _CHUNK, _CHUNK)],
                        acc_sh.at[idx_v.at[j]], sem, add=True))
                for d in descs:
                    d.wait()
            return carry

        lax.fori_loop(0, outer, body, 0)
        plsc.subcore_barrier()
        pltpu.sync_copy(
            acc_sh.at[pl.ds(sid * rpt, rpt)],
            out_hbm.at[cid, pl.ds(sid * rpt, rpt)])
        if tail:
            @pl.when(sid == _NS - 1)
            def _():
                pltpu.sync_copy(
                    acc_sh.at[pl.ds(_NS * rpt, tail)],
                    out_hbm.at[cid, pl.ds(_NS * rpt, tail)])

    return scat(e2d, dst3d, zeros_nc)


def _node_body(x_ref, agg_ref, agg2_ref, emb_ref, embW_ref, embb_ref,
               wa_ref, wb_ref, b0_ref, w1_ref, b1_ref,
               w2_ref, b2_ref, w3_ref, b3_ref, out_ref):
    c = jnp.dot(emb_ref[...], embW_ref[...],
                preferred_element_type=F32) + embb_ref[...]
    xc = x_ref[...] + c
    agg = agg_ref[...] + agg2_ref[...]
    h = (jnp.dot(xc.astype(BF16), wa_ref[...], preferred_element_type=F32)
         + jnp.dot(agg.astype(BF16), wb_ref[...], preferred_element_type=F32)
         + b0_ref[...])
    h = _elu(h)
    h = _elu(jnp.dot(h.astype(BF16), w1_ref[...],
                     preferred_element_type=F32) + b1_ref[...])
    h = _elu(jnp.dot(h.astype(BF16), w2_ref[...],
                     preferred_element_type=F32) + b2_ref[...])
    h = jnp.dot(h.astype(BF16), w3_ref[...],
                preferred_element_type=F32) + b3_ref[...]
    out_ref[...] = xc + h


def _node_mlp(x, agg, agg2, emb, emb_W, emb_b, nW0, nb0, nWh, nbh):
    N, C = x.shape
    T = _pick_tile(N, (2000, 1000, 500, 200, 100, 50, 40, 25, 16, 8))
    wa = nW0[0 * C:1 * C].astype(BF16)
    wb = nW0[1 * C:2 * C].astype(BF16)
    w1, w2, w3 = (nWh[i].astype(BF16) for i in range(3))
    b0 = nb0.reshape(1, C)
    b1, b2, b3 = (nbh[i].reshape(1, C) for i in range(3))

    row_spec = pl.BlockSpec((T, C), lambda i: (i, 0))
    full = lambda a: pl.BlockSpec(a.shape, lambda i: (0,) * a.ndim)
    return pl.pallas_call(
        _node_body,
        grid=(N // T,),
        in_specs=[row_spec, row_spec, row_spec,
                  full(emb), full(emb_W), full(b0),
                  full(wa), full(wb), full(b0),
                  full(w1), full(b1), full(w2), full(b2), full(w3), full(b3)],
        out_specs=row_spec,
        out_shape=jax.ShapeDtypeStruct((N, C), F32),
        compiler_params=pltpu.CompilerParams(
            dimension_semantics=("arbitrary",)),
    )(x, agg, agg2, emb, emb_W, emb_b.reshape(1, C),
      wa, wb, b0, w1, b1, w2, b2, w3, b3)


def kernel(x, e, emb, edge_index, edge_weight, halo_info, mask_send,
           mask_recv, buffer_send, buffer_recv, neighboring_procs, SIZE,
           emb_W, emb_b, eW0, eb0, eWh, ebh, nW0, nb0, nWh, nbh):
    src = edge_index[0]
    dst = edge_index[1]
    E = e.shape[0]
    N, C = x.shape
    u, v = _uv_project(x, eW0)
    ew2 = edge_weight.reshape(1, E)
    E2 = E // 2
    if E2 % (_KG * _CHUNK) == 0 and E2 % _pick_tile(E2, _EDGE_TILES) == 0:
        pre1 = _sc_gather_sum(u, v, src[:E2], dst[:E2])
        pre2 = _sc_gather_sum(u, v, src[E2:], dst[E2:])
        eo1 = _edge_mlp_slice(pre1, e, ew2, emb, emb_W, emb_b,
                              eW0, eb0, eWh, ebh, 0, None)
        e_out = _edge_mlp_slice(pre2, e, ew2, emb, emb_W, emb_b,
                                eW0, eb0, eWh, ebh, E2, eo1)
    else:
        pre = _sc_gather_sum(u, v, src, dst)
        e_out = _edge_mlp_slice(pre, e, ew2, emb, emb_W, emb_b,
                                eW0, eb0, eWh, ebh, 0, None)
    if E % (_CHUNK * _K) == 0:
        dst3d = dst.reshape(E // (_CHUNK * _K), _K, _CHUNK)
        zeros_nc = jnp.zeros((N, C), F32)
        partials = _sc_scatter_add(e_out, dst3d, zeros_nc)
        agg, agg2 = partials[0], partials[1]
    else:
        agg = jax.ops.segment_sum(e_out, dst, num_segments=N)
        agg2 = jnp.zeros((N, C), F32)
    x_out = _node_mlp(x, agg, agg2, emb, emb_W, emb_b, nW0, nb0, nWh, nbh)
    return (x_out, e_out)
